# baseline (speedup 1.0000x reference)
import jax
import jax.numpy as jnp
from jax.experimental import pallas as pl
from jax.experimental.pallas import tpu as pltpu

_T = 8192
_D = 768
_E = 8
_K = 2
_TM = 2048


def _moe_kernel(x_ref, wg_ref, we_ref, out_ref, ew_ref, xb_ref):
    e = pl.program_id(1)

    @pl.when(e == 0)
    def _gating():
        x = x_ref[...]
        g = jax.lax.dot_general(
            x, wg_ref[...], (((1,), (1,)), ((), ())),
            preferred_element_type=jnp.float32,
        )
        cols = jax.lax.broadcasted_iota(jnp.int32, (_TM, _E), 1)
        l1 = jnp.max(g, axis=1, keepdims=True)
        i1 = jnp.argmax(g, axis=1).reshape(_TM, 1)
        masked = jnp.where(cols == i1, -jnp.inf, g)
        l2 = jnp.max(masked, axis=1, keepdims=True)
        i2 = jnp.argmax(masked, axis=1).reshape(_TM, 1)
        e2 = jnp.exp(l2 - l1)
        w1 = 1.0 / (1.0 + e2)
        w2 = e2 / (1.0 + e2)
        ew_ref[...] = (jnp.where(cols == i1, w1, 0.0)
                       + jnp.where(cols == i2, w2, 0.0))
        xb_ref[...] = x.astype(jnp.bfloat16)

    y = jax.lax.dot_general(
        xb_ref[...], we_ref[0], (((1,), (1,)), ((), ())),
        preferred_element_type=jnp.float32,
    )
    ew_all = ew_ref[...]
    cols_e = jax.lax.broadcasted_iota(jnp.int32, (_TM, _E), 1)
    wcol = jnp.sum(jnp.where(cols_e == e, ew_all, 0.0), axis=1, keepdims=True)
    contrib = wcol * y

    @pl.when(e == 0)
    def _init():
        out_ref[...] = contrib

    @pl.when(e > 0)
    def _acc():
        out_ref[...] = out_ref[...] + contrib


def kernel(x, Wg, We):
    we_b = We.astype(jnp.bfloat16)
    out, ew = pl.pallas_call(
        _moe_kernel,
        grid=(_T // _TM, _E),
        in_specs=[
            pl.BlockSpec((_TM, _D), lambda i, e: (i, 0)),
            pl.BlockSpec((_E, _D), lambda i, e: (0, 0)),
            pl.BlockSpec((1, _D, _D), lambda i, e: (e, 0, 0)),
        ],
        out_specs=[
            pl.BlockSpec((_TM, _D), lambda i, e: (i, 0)),
            pl.BlockSpec((_TM, _E), lambda i, e: (i, 0)),
        ],
        out_shape=[
            jax.ShapeDtypeStruct((_T, _D), jnp.float32),
            jax.ShapeDtypeStruct((_T, _E), jnp.float32),
        ],
        scratch_shapes=[pltpu.VMEM((_TM, _D), jnp.bfloat16)],
    )(x, Wg, we_b)
    return (out, ew)


# speedup vs baseline: 1.3937x; 1.3937x over previous
import jax
import jax.numpy as jnp
from jax.experimental import pallas as pl
from jax.experimental.pallas import tpu as pltpu

_T = 8192
_D = 768
_E = 8
_K = 2
_TM = 1024


def _moe_kernel(x_ref, wg_ref, we_ref, out_ref, ew_ref, web_ref):
    @pl.when(pl.program_id(0) == 0)
    def _cast_weights():
        for e in range(_E):
            web_ref[e] = we_ref[e].astype(jnp.bfloat16)

    x = x_ref[...]
    g = jax.lax.dot_general(
        x, wg_ref[...], (((1,), (1,)), ((), ())),
        preferred_element_type=jnp.float32,
    )
    cols = jax.lax.broadcasted_iota(jnp.int32, (_TM, _E), 1)
    l1 = jnp.max(g, axis=1, keepdims=True)
    i1 = jnp.argmax(g, axis=1).reshape(_TM, 1)
    masked = jnp.where(cols == i1, -jnp.inf, g)
    l2 = jnp.max(masked, axis=1, keepdims=True)
    i2 = jnp.argmax(masked, axis=1).reshape(_TM, 1)
    e2 = jnp.exp(l2 - l1)
    w1 = 1.0 / (1.0 + e2)
    w2 = e2 / (1.0 + e2)
    ew = jnp.where(cols == i1, w1, 0.0) + jnp.where(cols == i2, w2, 0.0)
    ew_ref[...] = ew

    xb = x.astype(jnp.bfloat16)
    acc = jnp.zeros((_TM, _D), jnp.float32)
    for e in range(_E):
        y = jax.lax.dot_general(
            xb, web_ref[e], (((1,), (1,)), ((), ())),
            preferred_element_type=jnp.float32,
        )
        acc = acc + ew[:, e].reshape(_TM, 1) * y
    out_ref[...] = acc


def kernel(x, Wg, We):
    out, ew = pl.pallas_call(
        _moe_kernel,
        grid=(_T // _TM,),
        in_specs=[
            pl.BlockSpec((_TM, _D), lambda i: (i, 0)),
            pl.BlockSpec((_E, _D), lambda i: (0, 0)),
            pl.BlockSpec((_E, _D, _D), lambda i: (0, 0, 0)),
        ],
        out_specs=[
            pl.BlockSpec((_TM, _D), lambda i: (i, 0)),
            pl.BlockSpec((_TM, _E), lambda i: (i, 0)),
        ],
        out_shape=[
            jax.ShapeDtypeStruct((_T, _D), jnp.float32),
            jax.ShapeDtypeStruct((_T, _E), jnp.float32),
        ],
        scratch_shapes=[pltpu.VMEM((_E, _D, _D), jnp.bfloat16)],
    )(x, Wg, We)
    return (out, ew)
